# baseline (speedup 1.0000x reference)
import functools

import numpy as np

import jax
import jax.numpy as jnp
from jax import lax
from jax.experimental import pallas as pl
from jax.experimental.pallas import tpu as pltpu
from jax.experimental.pallas import tpu_sc as plsc

_P_T = 0.1
_P_C = 0.1
_T_SPAN = 10
_C_SPAN = 2


def _span(seed_mask, span):
    L = seed_mask.shape[-1]
    m = jnp.zeros_like(seed_mask)
    for k in range(span):
        m = m | jnp.pad(seed_mask, ((0, 0), (k, 0)))[:, :L]
    return m


def _mask(key, shape, p, span):
    seed = jax.random.uniform(key, shape) < p
    empty = ~jnp.any(seed, axis=1)
    seed = seed.at[:, 0].set(seed[:, 0] | empty)
    return _span(seed, span)


def _const_masks():
    mk = jax.random.key(1)
    mask_t = _mask(jax.random.fold_in(mk, 0), (8, 512), _P_T, _T_SPAN)
    mask_c = _mask(jax.random.fold_in(mk, 1), (8, 64), _P_C, _C_SPAN)
    return np.asarray(mask_t), np.asarray(mask_c)


_MT, _MC = _const_masks()

_NC = 2
_NS = 16
_NW = _NC * _NS
_W = 512
_ROWS = 65536
_RPW = _ROWS // _NW
_CH = 32
_NCH = _RPW // _CH
_NBUF = 4
_NSL = _W // 16


def _sc_select(xf, keep_a, fill_a, mc_i, crep16):
    mesh = plsc.VectorSubcoreMesh(
        core_axis_name="c", subcore_axis_name="s",
        num_cores=_NC, num_subcores=_NS)

    @functools.partial(
        pl.kernel,
        out_type=jax.ShapeDtypeStruct((_ROWS, _W), jnp.float32),
        mesh=mesh,
        scratch_types=(
            [pltpu.VMEM((_CH, _W), jnp.float32)] * _NBUF
            + [pltpu.SemaphoreType.DMA] * (2 * _NBUF)
            + [
                pltpu.VMEM((_W,), jnp.float32),
                pltpu.VMEM((_W,), jnp.float32),
                pltpu.VMEM((64, 16), jnp.int32),
                pltpu.VMEM((16,), jnp.float32),
                pltpu.SemaphoreType.DMA,
            ]
        ),
    )
    def k(x_hbm, ka_hbm, fa_hbm, mc_hbm, cr_hbm, o_hbm,
          b0, b1, b2, b3, i0, i1, i2, i3, o0, o1, o2, o3,
          keep_v, fill_v, mc_v, cr_v, aux_sem):
        bufs = [b0, b1, b2, b3]
        sin = [i0, i1, i2, i3]
        sout = [o0, o1, o2, o3]
        wid = lax.axis_index("s") * _NC + lax.axis_index("c")
        base = wid * _RPW
        b = wid // 4

        pltpu.async_copy(ka_hbm.at[b], keep_v, aux_sem).wait()
        pltpu.async_copy(fa_hbm.at[b], fill_v, aux_sem).wait()
        pltpu.async_copy(mc_hbm.at[b], mc_v, aux_sem).wait()
        pltpu.async_copy(cr_hbm, cr_v, aux_sem).wait()

        def in_desc(kk, q):
            return pltpu.make_async_copy(
                x_hbm.at[pl.ds(base + kk * _CH, _CH)], bufs[q], sin[q])

        def out_desc(kk, q):
            return pltpu.make_async_copy(
                bufs[q], o_hbm.at[pl.ds(base + kk * _CH, _CH)], sout[q])

        def compute(q, kk):
            buf = bufs[q]
            crv = cr_v[...]
            h0 = (kk * _CH) % 64

            for half in range(2):
                off0 = half * (_W // 2)
                kvs = [keep_v[pl.ds(off0 + j * 16, 16)] for j in range(_NSL // 2)]
                fvs = [fill_v[pl.ds(off0 + j * 16, 16)] for j in range(_NSL // 2)]

                def row_body(i, carry, off0=off0, kvs=kvs, fvs=fvs):
                    cfv = mc_v[h0 + i, pl.ds(0, 16)] != 0
                    for j in range(_NSL // 2):
                        off = off0 + j * 16
                        xv = buf[i, pl.ds(off, 16)]
                        val = xv * kvs[j] + fvs[j]
                        buf[i, pl.ds(off, 16)] = jnp.where(cfv, crv, val)
                    return carry

                lax.fori_loop(0, _CH, row_body, None)

        def chunk(kk, q, first, last):
            if not first:
                out_desc(kk - 2, (q + 2) % _NBUF).wait()
            if not last:
                in_desc(kk + 2, (q + 2) % _NBUF).start()
            in_desc(kk, q).wait()
            compute(q, kk)
            out_desc(kk, q).start()

        in_desc(0, 0).start()
        in_desc(1, 1).start()
        chunk(0, 0, True, False)
        chunk(1, 1, True, False)

        def body(g, carry):
            kk = 2 + g * 4
            for par in range(4):
                chunk(kk + par, (2 + par) % _NBUF, False, False)
            return carry

        lax.fori_loop(0, (_NCH - 4) // 4, body, None)

        chunk(_NCH - 2, (_NCH - 2) % _NBUF, False, True)
        chunk(_NCH - 1, (_NCH - 1) % _NBUF, False, True)
        out_desc(_NCH - 2, (_NCH - 2) % _NBUF).wait()
        out_desc(_NCH - 1, (_NCH - 1) % _NBUF).wait()

    return k(xf, keep_a, fill_a, mc_i, crep16)


def kernel(x, t_mask_replacement, c_mask_replacement):
    B, D, H, W = x.shape
    xf = x.reshape(_ROWS, _W)
    mt_f = jnp.asarray(_MT.astype(np.float32))
    keep_a = 1.0 - mt_f
    fill_a = mt_f * t_mask_replacement.astype(jnp.float32)
    mc_i = jnp.asarray(
        np.broadcast_to(_MC[:, :, None], (8, 64, 16)).astype(np.int32))
    crep16 = jnp.full((16,), c_mask_replacement, jnp.float32)
    out = _sc_select(xf, keep_a, fill_a, mc_i, crep16).reshape(B, D, H, W)
    mask_t = jnp.asarray(_MT)
    mask_c = jnp.asarray(_MC)
    return (out, x, mask_t, mask_c)


# speedup vs baseline: 1.2081x; 1.2081x over previous
import numpy as np

import jax
import jax.numpy as jnp
from jax.experimental import pallas as pl
from jax.experimental.pallas import tpu as pltpu

_P_T = 0.1
_P_C = 0.1
_T_SPAN = 10
_C_SPAN = 2


def _span(seed_mask, span):
    L = seed_mask.shape[-1]
    m = jnp.zeros_like(seed_mask)
    for k in range(span):
        m = m | jnp.pad(seed_mask, ((0, 0), (k, 0)))[:, :L]
    return m


def _mask(key, shape, p, span):
    seed = jax.random.uniform(key, shape) < p
    empty = ~jnp.any(seed, axis=1)
    seed = seed.at[:, 0].set(seed[:, 0] | empty)
    return _span(seed, span)


def _const_masks():
    mk = jax.random.key(1)
    mask_t = _mask(jax.random.fold_in(mk, 0), (8, 512), _P_T, _T_SPAN)
    mask_c = _mask(jax.random.fold_in(mk, 1), (8, 64), _P_C, _C_SPAN)
    return np.asarray(mask_t), np.asarray(mask_c)


_MT, _MC = _const_masks()
_SEL = np.where(
    _MC[:, :, None], np.int8(2), np.where(_MT[:, None, :], np.int8(1), np.int8(0))
)


def _body(reps_ref, sel_ref, x_ref, o_ref):
    t = reps_ref[0]
    c = reps_ref[1]
    s = sel_ref[...]
    o = jnp.where(s == 1, t, x_ref[...])
    o_ref[...] = jnp.where(s == 2, c, o)


def kernel(x, t_mask_replacement, c_mask_replacement):
    B, D, H, W = x.shape
    reps = jnp.stack([t_mask_replacement, c_mask_replacement]).astype(x.dtype)
    sel = jnp.asarray(_SEL)
    dblk = 64
    out = pl.pallas_call(
        _body,
        grid=(B, D // dblk),
        in_specs=[
            pl.BlockSpec(memory_space=pltpu.SMEM),
            pl.BlockSpec((1, H, W), lambda b, i: (b, 0, 0)),
            pl.BlockSpec((1, dblk, H, W), lambda b, i: (b, i, 0, 0)),
        ],
        out_specs=pl.BlockSpec((1, dblk, H, W), lambda b, i: (b, i, 0, 0)),
        out_shape=jax.ShapeDtypeStruct(x.shape, x.dtype),
    )(reps, sel, x)
    mask_t = jnp.asarray(_MT)
    mask_c = jnp.asarray(_MC)
    return (out, x, mask_t, mask_c)
